# baseline (speedup 1.0000x reference)
import functools

import jax
import jax.numpy as jnp
from jax import lax
from jax.experimental import pallas as pl
from jax.experimental.pallas import tpu as pltpu
from jax.experimental.pallas import tpu_sc as plsc

B = 16
N = 16384
NSKIP = 65536
D_IN = 64
D_HID = 64
D_OUT = 128
D_SKIP = 64

BK1 = 2048
BK2 = 4096


def _bounds_rows(ids, total):
    cols = [
        jnp.full((1, 1), jnp.sum((ids < s).astype(jnp.int32)), jnp.int32)
        for s in range(1, B)
    ]
    lt = jnp.concatenate([jnp.zeros((1, 1), jnp.int32)] + cols, axis=1)
    le = jnp.concatenate(cols + [jnp.full((1, 1), total, jnp.int32)], axis=1)
    return lt, le


def _mlp_segmax_body(xb, posb, ball, w1a, w1b, b1r, w2r, b2r, out_ref, bnd):
    h1 = jnp.dot(xb[...], w1a[...], preferred_element_type=jnp.float32)
    h1 = h1 + jnp.dot(posb[...], w1b[...], preferred_element_type=jnp.float32)
    h1 = jnp.maximum(h1 + b1r[...][0][None, :], 0.0)
    h = jnp.dot(h1, w2r[...], preferred_element_type=jnp.float32)
    h = jnp.maximum(h + b2r[...][0][None, :], 0.0)

    @pl.when(pl.program_id(0) == 0)
    def _():
        lt, le = _bounds_rows(ball[...], N)
        bnd[0:1, :] = lt
        bnd[1:2, :] = le
        out_ref[...] = jnp.full((B, D_OUT), -jnp.inf, jnp.float32)

    r_g = (lax.broadcasted_iota(jnp.int32, (BK1, B), 0)
           + pl.program_id(0) * BK1)
    m_all = (r_g >= bnd[0:1, :]) & (r_g < bnd[1:2, :])

    rows = [
        jnp.max(jnp.where(m_all[:, s:s + 1], h, -jnp.inf), axis=0, keepdims=True)
        for s in range(B)
    ]
    out_ref[...] = jnp.maximum(out_ref[...], jnp.concatenate(rows, axis=0))


def _mlp_segmax(x, pos, batch, W1, b1, W2, b2):
    w1a = W1[:D_IN]
    w1b = W1[D_IN:]
    batc = batch.astype(jnp.int32).reshape(N // 128, 128)
    grid = (N // BK1,)
    return pl.pallas_call(
        _mlp_segmax_body,
        grid=grid,
        in_specs=[
            pl.BlockSpec((BK1, D_IN), lambda i: (i, 0)),
            pl.BlockSpec((BK1, 3), lambda i: (i, 0)),
            pl.BlockSpec((N // 128, 128), lambda i: (0, 0)),
            pl.BlockSpec((D_IN, D_HID), lambda i: (0, 0)),
            pl.BlockSpec((3, D_HID), lambda i: (0, 0)),
            pl.BlockSpec((1, D_HID), lambda i: (0, 0)),
            pl.BlockSpec((D_HID, D_OUT), lambda i: (0, 0)),
            pl.BlockSpec((1, D_OUT), lambda i: (0, 0)),
        ],
        out_specs=pl.BlockSpec((B, D_OUT), lambda i: (0, 0)),
        out_shape=jax.ShapeDtypeStruct((B, D_OUT), jnp.float32),
        scratch_shapes=[pltpu.VMEM((2, B), jnp.int32)],
    )(x, pos, batc, w1a, w1b, b1.reshape(1, D_HID), W2, b2.reshape(1, D_OUT))


def _assemble_body(pooled_ref, bskall, xsk_ref, out_ref, bnd):
    @pl.when(pl.program_id(0) == 0)
    def _():
        lt, le = _bounds_rows(bskall[...], NSKIP)
        bnd[0:1, :] = lt
        bnd[1:2, :] = le

    r_g = (lax.broadcasted_iota(jnp.int32, (BK2, B), 0)
           + pl.program_id(0) * BK2)
    onehot = ((r_g >= bnd[0:1, :]) & (r_g < bnd[1:2, :])).astype(jnp.float32)
    g = jnp.dot(onehot, pooled_ref[...], preferred_element_type=jnp.float32)
    g = jnp.where(g < -0.5, -jnp.inf, g)
    out_ref[:, :D_OUT] = g
    out_ref[:, D_OUT:] = xsk_ref[...]


def _assemble(pooled, batch_skip, x_skip):
    bskc = batch_skip.astype(jnp.int32).reshape(NSKIP // 128, 128)
    grid = (NSKIP // BK2,)
    return pl.pallas_call(
        _assemble_body,
        grid=grid,
        in_specs=[
            pl.BlockSpec((B, D_OUT), lambda i: (0, 0)),
            pl.BlockSpec((NSKIP // 128, 128), lambda i: (0, 0)),
            pl.BlockSpec((BK2, D_SKIP), lambda i: (i, 0)),
        ],
        out_specs=pl.BlockSpec((BK2, D_OUT + D_SKIP), lambda i: (i, 0)),
        out_shape=jax.ShapeDtypeStruct((NSKIP, D_OUT + D_SKIP), jnp.float32),
        scratch_shapes=[pltpu.VMEM((2, B), jnp.int32)],
    )(pooled, bskc, x_skip)


_NC, _NS = 2, 16
_NW = _NC * _NS
_CHUNK = NSKIP // _NW
_SUB = 256
_NSUB = _CHUNK // _SUB


def _sc_assemble(pooled, bsk, x_skip):
    mesh = plsc.VectorSubcoreMesh(core_axis_name="c", subcore_axis_name="s")

    @functools.partial(
        pl.kernel,
        out_type=jax.ShapeDtypeStruct((NSKIP, D_OUT + D_SKIP), jnp.float32),
        mesh=mesh,
        scratch_types=[
            pltpu.VMEM((_CHUNK // 128, 128), jnp.int32),
            pltpu.VMEM((_SUB, D_OUT), jnp.float32),
            pltpu.VMEM((_SUB, D_SKIP), jnp.float32),
            pltpu.VMEM((B, D_OUT), jnp.float32),
            pltpu.SemaphoreType.DMA,
        ],
    )
    def k(pooled_hbm, bsk_hbm, xsk_hbm, out_hbm, idx_v, gath_v, xsb_v,
          pooled_v, sem):
        wid = lax.axis_index("s") * _NC + lax.axis_index("c")
        base = wid * _CHUNK
        pltpu.sync_copy(pooled_hbm, pooled_v)
        for j in range(_NSUB):
            b = base + j * _SUB
            for k in range(_SUB // 128):
                r = j * (_SUB // 128) + k
                pltpu.sync_copy(bsk_hbm.at[pl.ds(b + k * 128, 128)],
                                idx_v.at[r])
                pltpu.async_copy(pooled_hbm.at[idx_v.at[r]],
                                 gath_v.at[pl.ds(k * 128, 128)], sem).wait()
            pltpu.sync_copy(xsk_hbm.at[pl.ds(b, _SUB)], xsb_v)
            pltpu.sync_copy(gath_v, out_hbm.at[pl.ds(b, _SUB), pl.ds(0, D_OUT)])
            pltpu.sync_copy(xsb_v, out_hbm.at[pl.ds(b, _SUB), pl.ds(D_OUT, D_SKIP)])

    return k(pooled, bsk, x_skip)


def kernel(x, pos, batch, x_skip, pos_skip, batch_skip, W1, b1, W2, b2):
    pooled = _mlp_segmax(x, pos, batch, W1, b1, W2, b2)
    out = _sc_assemble(pooled, batch_skip.astype(jnp.int32), x_skip)
    return (out, pos_skip, batch_skip)


# speedup vs baseline: 2.3132x; 2.3132x over previous
import functools

import jax
import jax.numpy as jnp
from jax import lax
from jax.experimental import pallas as pl
from jax.experimental.pallas import tpu as pltpu
from jax.experimental.pallas import tpu_sc as plsc

B = 16
N = 16384
NSKIP = 65536
D_IN = 64
D_HID = 64
D_OUT = 128
D_SKIP = 64

BK1 = 2048
BK2 = 4096


def _bounds_rows(ids, total):
    cols = [
        jnp.full((1, 1), jnp.sum((ids < s).astype(jnp.int32)), jnp.int32)
        for s in range(1, B)
    ]
    lt = jnp.concatenate([jnp.zeros((1, 1), jnp.int32)] + cols, axis=1)
    le = jnp.concatenate(cols + [jnp.full((1, 1), total, jnp.int32)], axis=1)
    return lt, le


def _mlp_segmax_body(xb, posb, ball, w1a, w1b, b1r, w2r, b2r, out_ref, bnd):
    h1 = jnp.dot(xb[...], w1a[...], preferred_element_type=jnp.float32)
    h1 = h1 + jnp.dot(posb[...], w1b[...], preferred_element_type=jnp.float32)
    h1 = jnp.maximum(h1 + b1r[...][0][None, :], 0.0)
    h = jnp.dot(h1, w2r[...], preferred_element_type=jnp.float32)
    h = jnp.maximum(h + b2r[...][0][None, :], 0.0)

    @pl.when(pl.program_id(0) == 0)
    def _():
        lt, le = _bounds_rows(ball[...], N)
        bnd[0:1, :] = lt
        bnd[1:2, :] = le
        out_ref[...] = jnp.full((B, D_OUT), -jnp.inf, jnp.float32)

    r_g = (lax.broadcasted_iota(jnp.int32, (BK1, B), 0)
           + pl.program_id(0) * BK1)
    m_all = (r_g >= bnd[0:1, :]) & (r_g < bnd[1:2, :])

    rows = [
        jnp.max(jnp.where(m_all[:, s:s + 1], h, -jnp.inf), axis=0, keepdims=True)
        for s in range(B)
    ]
    out_ref[...] = jnp.maximum(out_ref[...], jnp.concatenate(rows, axis=0))


def _mlp_segmax(x, pos, batch, W1, b1, W2, b2):
    w1a = W1[:D_IN]
    w1b = W1[D_IN:]
    batc = batch.astype(jnp.int32).reshape(N // 128, 128)
    grid = (N // BK1,)
    return pl.pallas_call(
        _mlp_segmax_body,
        grid=grid,
        in_specs=[
            pl.BlockSpec((BK1, D_IN), lambda i: (i, 0)),
            pl.BlockSpec((BK1, 3), lambda i: (i, 0)),
            pl.BlockSpec((N // 128, 128), lambda i: (0, 0)),
            pl.BlockSpec((D_IN, D_HID), lambda i: (0, 0)),
            pl.BlockSpec((3, D_HID), lambda i: (0, 0)),
            pl.BlockSpec((1, D_HID), lambda i: (0, 0)),
            pl.BlockSpec((D_HID, D_OUT), lambda i: (0, 0)),
            pl.BlockSpec((1, D_OUT), lambda i: (0, 0)),
        ],
        out_specs=pl.BlockSpec((B, D_OUT), lambda i: (0, 0)),
        out_shape=jax.ShapeDtypeStruct((B, D_OUT), jnp.float32),
        scratch_shapes=[pltpu.VMEM((2, B), jnp.int32)],
    )(x, pos, batc, w1a, w1b, b1.reshape(1, D_HID), W2, b2.reshape(1, D_OUT))


def _assemble_body(pooled_ref, bskall, xsk_ref, out_ref, bnd):
    @pl.when(pl.program_id(0) == 0)
    def _():
        lt, le = _bounds_rows(bskall[...], NSKIP)
        bnd[0:1, :] = lt
        bnd[1:2, :] = le

    r_g = (lax.broadcasted_iota(jnp.int32, (BK2, B), 0)
           + pl.program_id(0) * BK2)
    onehot = ((r_g >= bnd[0:1, :]) & (r_g < bnd[1:2, :])).astype(jnp.float32)
    g = jnp.dot(onehot, pooled_ref[...], preferred_element_type=jnp.float32)
    g = jnp.where(g < -0.5, -jnp.inf, g)
    out_ref[:, :D_OUT] = g
    out_ref[:, D_OUT:] = xsk_ref[...]


def _assemble(pooled, batch_skip, x_skip):
    bskc = batch_skip.astype(jnp.int32).reshape(NSKIP // 128, 128)
    grid = (NSKIP // BK2,)
    return pl.pallas_call(
        _assemble_body,
        grid=grid,
        in_specs=[
            pl.BlockSpec((B, D_OUT), lambda i: (0, 0)),
            pl.BlockSpec((NSKIP // 128, 128), lambda i: (0, 0)),
            pl.BlockSpec((BK2, D_SKIP), lambda i: (i, 0)),
        ],
        out_specs=pl.BlockSpec((BK2, D_OUT + D_SKIP), lambda i: (i, 0)),
        out_shape=jax.ShapeDtypeStruct((NSKIP, D_OUT + D_SKIP), jnp.float32),
        scratch_shapes=[pltpu.VMEM((2, B), jnp.int32)],
    )(pooled, bskc, x_skip)


_NC, _NS = 2, 16
_NW = _NC * _NS
_CHUNK = NSKIP // _NW
_SUB = 128
_NSUB = _CHUNK // _SUB
_NBUF = 2
_GPS = _SUB // 128


def _sc_assemble(pooled, bsk2d, x_skip):
    mesh = plsc.VectorSubcoreMesh(core_axis_name="c", subcore_axis_name="s")

    @functools.partial(
        pl.kernel,
        out_type=jax.ShapeDtypeStruct((NSKIP, D_OUT + D_SKIP), jnp.float32),
        mesh=mesh,
        scratch_types=[
            pltpu.VMEM((_CHUNK // 128, 128), jnp.int32),
            pltpu.VMEM((_NBUF, _SUB, D_OUT), jnp.float32),
            pltpu.VMEM((_NBUF, _SUB, D_SKIP), jnp.float32),
            pltpu.VMEM_SHARED((B, D_OUT), jnp.float32),
            pltpu.SemaphoreType.DMA((_NBUF,)),
            pltpu.SemaphoreType.DMA((_NBUF,)),
            pltpu.SemaphoreType.DMA((_NBUF,)),
        ],
    )
    def k(pooled_hbm, bsk_hbm, xsk_hbm, out_hbm, idx_v, gath_v, xsb_v,
          pooled_sh, gsem, xsem, wsem):
        cid = lax.axis_index("c")
        sid = lax.axis_index("s")
        wid = sid * _NC + cid
        base = wid * _CHUNK

        @pl.when(sid == 0)
        def _():
            pltpu.sync_copy(pooled_hbm, pooled_sh)

        plsc.subcore_barrier()
        pltpu.sync_copy(
            bsk_hbm.at[pl.ds(wid * (_CHUNK // 128), _CHUNK // 128)], idx_v)

        def start(j):
            bi = j % _NBUF
            b = base + j * _SUB
            for g in range(_GPS):
                r = j * _GPS + g
                pltpu.async_copy(
                    pooled_sh.at[idx_v.at[r]],
                    gath_v.at[bi, pl.ds(g * 128, 128)], gsem.at[bi])
            pltpu.async_copy(xsk_hbm.at[pl.ds(b, _SUB)], xsb_v.at[bi],
                             xsem.at[bi])

        def drain(j):
            bi = j % _NBUF
            b = base + j * _SUB
            for g in range(_GPS):
                r = j * _GPS + g
                pltpu.make_async_copy(
                    pooled_sh.at[idx_v.at[r]],
                    gath_v.at[bi, pl.ds(g * 128, 128)], gsem.at[bi]).wait()
            pltpu.make_async_copy(xsk_hbm.at[pl.ds(b, _SUB)], xsb_v.at[bi],
                                  xsem.at[bi]).wait()
            pltpu.async_copy(gath_v.at[bi],
                             out_hbm.at[pl.ds(b, _SUB), pl.ds(0, D_OUT)],
                             wsem.at[bi])
            pltpu.async_copy(xsb_v.at[bi],
                             out_hbm.at[pl.ds(b, _SUB), pl.ds(D_OUT, D_SKIP)],
                             wsem.at[bi])

        def wait_writes(j):
            bi = j % _NBUF
            b = base + j * _SUB
            pltpu.make_async_copy(
                gath_v.at[bi],
                out_hbm.at[pl.ds(b, _SUB), pl.ds(0, D_OUT)],
                wsem.at[bi]).wait()
            pltpu.make_async_copy(
                xsb_v.at[bi],
                out_hbm.at[pl.ds(b, _SUB), pl.ds(D_OUT, D_SKIP)],
                wsem.at[bi]).wait()

        start(0)
        for j in range(_NSUB):
            if j + 1 < _NSUB:
                if j + 1 >= _NBUF:
                    wait_writes(j + 1 - _NBUF)
                start(j + 1)
            drain(j)
        for j in range(_NSUB - _NBUF + 1, _NSUB):
            wait_writes(j)
        wait_writes(_NSUB - _NBUF)

    return k(pooled, bsk2d, x_skip)


def kernel(x, pos, batch, x_skip, pos_skip, batch_skip, W1, b1, W2, b2):
    pooled = _mlp_segmax(x, pos, batch, W1, b1, W2, b2)
    bsk2d = batch_skip.astype(jnp.int32).reshape(NSKIP // 128, 128)
    out = _sc_assemble(pooled, bsk2d, x_skip)
    return (out, pos_skip, batch_skip)


# speedup vs baseline: 2.7410x; 1.1849x over previous
import functools

import jax
import jax.numpy as jnp
from jax import lax
from jax.experimental import pallas as pl
from jax.experimental.pallas import tpu as pltpu
from jax.experimental.pallas import tpu_sc as plsc

B = 16
N = 16384
NSKIP = 65536
D_IN = 64
D_HID = 64
D_OUT = 128
D_SKIP = 64

BK1 = 2048
BK2 = 4096


def _bounds_rows(ids, total):
    cols = [
        jnp.full((1, 1), jnp.sum((ids < s).astype(jnp.int32)), jnp.int32)
        for s in range(1, B)
    ]
    lt = jnp.concatenate([jnp.zeros((1, 1), jnp.int32)] + cols, axis=1)
    le = jnp.concatenate(cols + [jnp.full((1, 1), total, jnp.int32)], axis=1)
    return lt, le


def _mlp_segmax_body(xb, posb, ball, w1a, w1b, b1r, w2r, b2r, out_ref,
                     bnd, bnd_sm, hbuf):
    h1 = jnp.dot(xb[...], w1a[...], preferred_element_type=jnp.float32)
    h1 = h1 + jnp.dot(posb[...], w1b[...], preferred_element_type=jnp.float32)
    h1 = jnp.maximum(h1 + b1r[...][0][None, :], 0.0)
    h = jnp.dot(h1, w2r[...], preferred_element_type=jnp.float32)
    h = jnp.maximum(h + b2r[...][0][None, :], 0.0)

    @pl.when(pl.program_id(0) == 0)
    def _():
        ids = ball[...]
        lts = [jnp.sum((ids < s).astype(jnp.int32)) for s in range(1, B)]
        cols = [jnp.full((1, 1), v, jnp.int32) for v in lts]
        lt = jnp.concatenate([jnp.zeros((1, 1), jnp.int32)] + cols, axis=1)
        le = jnp.concatenate(cols + [jnp.full((1, 1), N, jnp.int32)], axis=1)
        bnd[0:1, :] = lt
        bnd[1:2, :] = le
        for s in range(B):
            bnd_sm[0, s] = jnp.int32(0) if s == 0 else lts[s - 1]
            bnd_sm[1, s] = jnp.int32(N) if s == B - 1 else lts[s]
        out_ref[...] = jnp.full((B, D_OUT), -jnp.inf, jnp.float32)

    hbuf[...] = h
    base = pl.program_id(0) * BK1

    hgf = jnp.max(h.reshape(BK1 // 64, 8, 8, D_OUT), axis=1)
    hgf = hgf.reshape(BK1 // 8, D_OUT)

    llo_row = jnp.clip(bnd[0:1, :] - base, 0, BK1)
    lhi_row = jnp.clip(bnd[1:2, :] - base, 0, BK1)
    glo8 = ((llo_row + 63) >> 6) << 3
    ghi8 = (lhi_row >> 6) << 3
    rr = lax.broadcasted_iota(jnp.int32, (BK1 // 8, B), 0)
    m_g = (rr >= glo8) & (rr < ghi8)

    i64 = lax.broadcasted_iota(jnp.int32, (64, 1), 0)
    rows = []
    for s in range(B):
        gmax = jnp.max(jnp.where(m_g[:, s:s + 1], hgf, -jnp.inf), axis=0,
                       keepdims=True)
        llo = jnp.clip(bnd_sm[0, s] - base, 0, BK1)
        lhi = jnp.clip(bnd_sm[1, s] - base, 0, BK1)
        slo = jnp.minimum(llo, BK1 - 64)
        shi = jnp.clip(lhi - 64, 0, BK1 - 64)
        elo = hbuf[pl.ds(slo, 64), :]
        mlo = ((i64 + slo) >= llo) & ((i64 + slo) < lhi)
        emax = jnp.max(jnp.where(mlo, elo, -jnp.inf), axis=0, keepdims=True)
        ehi = hbuf[pl.ds(shi, 64), :]
        mhi = ((i64 + shi) >= llo) & ((i64 + shi) < lhi)
        emax = jnp.maximum(
            emax,
            jnp.max(jnp.where(mhi, ehi, -jnp.inf), axis=0, keepdims=True))
        rows.append(jnp.maximum(gmax, emax))
    out_ref[...] = jnp.maximum(out_ref[...], jnp.concatenate(rows, axis=0))


def _mlp_segmax(x, pos, batch, W1, b1, W2, b2):
    w1a = W1[:D_IN]
    w1b = W1[D_IN:]
    batc = batch.astype(jnp.int32).reshape(N // 128, 128)
    grid = (N // BK1,)
    return pl.pallas_call(
        _mlp_segmax_body,
        grid=grid,
        in_specs=[
            pl.BlockSpec((BK1, D_IN), lambda i: (i, 0)),
            pl.BlockSpec((BK1, 3), lambda i: (i, 0)),
            pl.BlockSpec((N // 128, 128), lambda i: (0, 0)),
            pl.BlockSpec((D_IN, D_HID), lambda i: (0, 0)),
            pl.BlockSpec((3, D_HID), lambda i: (0, 0)),
            pl.BlockSpec((1, D_HID), lambda i: (0, 0)),
            pl.BlockSpec((D_HID, D_OUT), lambda i: (0, 0)),
            pl.BlockSpec((1, D_OUT), lambda i: (0, 0)),
        ],
        out_specs=pl.BlockSpec((B, D_OUT), lambda i: (0, 0)),
        out_shape=jax.ShapeDtypeStruct((B, D_OUT), jnp.float32),
        scratch_shapes=[pltpu.VMEM((2, B), jnp.int32),
                        pltpu.SMEM((2, B), jnp.int32),
                        pltpu.VMEM((BK1, D_OUT), jnp.float32)],
    )(x, pos, batc, w1a, w1b, b1.reshape(1, D_HID), W2, b2.reshape(1, D_OUT))


def _assemble_body(pooled_ref, bskall, xsk_ref, out_ref, bnd):
    @pl.when(pl.program_id(0) == 0)
    def _():
        lt, le = _bounds_rows(bskall[...], NSKIP)
        bnd[0:1, :] = lt
        bnd[1:2, :] = le

    r_g = (lax.broadcasted_iota(jnp.int32, (BK2, B), 0)
           + pl.program_id(0) * BK2)
    onehot = ((r_g >= bnd[0:1, :]) & (r_g < bnd[1:2, :])).astype(jnp.float32)
    g = jnp.dot(onehot, pooled_ref[...], preferred_element_type=jnp.float32)
    g = jnp.where(g < -0.5, -jnp.inf, g)
    out_ref[:, :D_OUT] = g
    out_ref[:, D_OUT:] = xsk_ref[...]


def _assemble(pooled, batch_skip, x_skip):
    bskc = batch_skip.astype(jnp.int32).reshape(NSKIP // 128, 128)
    grid = (NSKIP // BK2,)
    return pl.pallas_call(
        _assemble_body,
        grid=grid,
        in_specs=[
            pl.BlockSpec((B, D_OUT), lambda i: (0, 0)),
            pl.BlockSpec((NSKIP // 128, 128), lambda i: (0, 0)),
            pl.BlockSpec((BK2, D_SKIP), lambda i: (i, 0)),
        ],
        out_specs=pl.BlockSpec((BK2, D_OUT + D_SKIP), lambda i: (i, 0)),
        out_shape=jax.ShapeDtypeStruct((NSKIP, D_OUT + D_SKIP), jnp.float32),
        scratch_shapes=[pltpu.VMEM((2, B), jnp.int32)],
    )(pooled, bskc, x_skip)


_NC, _NS = 2, 16
_NW = _NC * _NS
_CHUNK = NSKIP // _NW
_SUB = 128
_NSUB = _CHUNK // _SUB
_NBUF = 2
_GPS = _SUB // 128


def _sc_assemble(pooled, bsk2d, x_skip):
    mesh = plsc.VectorSubcoreMesh(core_axis_name="c", subcore_axis_name="s")

    @functools.partial(
        pl.kernel,
        out_type=jax.ShapeDtypeStruct((NSKIP, D_OUT + D_SKIP), jnp.float32),
        mesh=mesh,
        scratch_types=[
            pltpu.VMEM((_CHUNK // 128, 128), jnp.int32),
            pltpu.VMEM((_NBUF, _SUB, D_OUT), jnp.float32),
            pltpu.VMEM((_NBUF, _SUB, D_SKIP), jnp.float32),
            pltpu.VMEM_SHARED((B, D_OUT), jnp.float32),
            pltpu.SemaphoreType.DMA((_NBUF,)),
            pltpu.SemaphoreType.DMA((_NBUF,)),
            pltpu.SemaphoreType.DMA((_NBUF,)),
        ],
    )
    def k(pooled_hbm, bsk_hbm, xsk_hbm, out_hbm, idx_v, gath_v, xsb_v,
          pooled_sh, gsem, xsem, wsem):
        cid = lax.axis_index("c")
        sid = lax.axis_index("s")
        wid = sid * _NC + cid
        base = wid * _CHUNK

        @pl.when(sid == 0)
        def _():
            pltpu.sync_copy(pooled_hbm, pooled_sh)

        plsc.subcore_barrier()
        pltpu.sync_copy(
            bsk_hbm.at[pl.ds(wid * (_CHUNK // 128), _CHUNK // 128)], idx_v)

        def start(j):
            bi = j % _NBUF
            b = base + j * _SUB
            for g in range(_GPS):
                r = j * _GPS + g
                pltpu.async_copy(
                    pooled_sh.at[idx_v.at[r]],
                    gath_v.at[bi, pl.ds(g * 128, 128)], gsem.at[bi])
            pltpu.async_copy(xsk_hbm.at[pl.ds(b, _SUB)], xsb_v.at[bi],
                             xsem.at[bi])

        def drain(j):
            bi = j % _NBUF
            b = base + j * _SUB
            for g in range(_GPS):
                r = j * _GPS + g
                pltpu.make_async_copy(
                    pooled_sh.at[idx_v.at[r]],
                    gath_v.at[bi, pl.ds(g * 128, 128)], gsem.at[bi]).wait()
            pltpu.make_async_copy(xsk_hbm.at[pl.ds(b, _SUB)], xsb_v.at[bi],
                                  xsem.at[bi]).wait()
            pltpu.async_copy(gath_v.at[bi],
                             out_hbm.at[pl.ds(b, _SUB), pl.ds(0, D_OUT)],
                             wsem.at[bi])
            pltpu.async_copy(xsb_v.at[bi],
                             out_hbm.at[pl.ds(b, _SUB), pl.ds(D_OUT, D_SKIP)],
                             wsem.at[bi])

        def wait_writes(j):
            bi = j % _NBUF
            b = base + j * _SUB
            pltpu.make_async_copy(
                gath_v.at[bi],
                out_hbm.at[pl.ds(b, _SUB), pl.ds(0, D_OUT)],
                wsem.at[bi]).wait()
            pltpu.make_async_copy(
                xsb_v.at[bi],
                out_hbm.at[pl.ds(b, _SUB), pl.ds(D_OUT, D_SKIP)],
                wsem.at[bi]).wait()

        start(0)
        for j in range(_NSUB):
            if j + 1 < _NSUB:
                if j + 1 >= _NBUF:
                    wait_writes(j + 1 - _NBUF)
                start(j + 1)
            drain(j)
        for j in range(_NSUB - _NBUF + 1, _NSUB):
            wait_writes(j)
        wait_writes(_NSUB - _NBUF)

    return k(pooled, bsk2d, x_skip)


def kernel(x, pos, batch, x_skip, pos_skip, batch_skip, W1, b1, W2, b2):
    pooled = _mlp_segmax(x, pos, batch, W1, b1, W2, b2)
    bsk2d = batch_skip.astype(jnp.int32).reshape(NSKIP // 128, 128)
    out = _sc_assemble(pooled, bsk2d, x_skip)
    return (out, pos_skip, batch_skip)


# speedup vs baseline: 2.7600x; 1.0069x over previous
import functools

import jax
import jax.numpy as jnp
from jax import lax
from jax.experimental import pallas as pl
from jax.experimental.pallas import tpu as pltpu
from jax.experimental.pallas import tpu_sc as plsc

B = 16
N = 16384
NSKIP = 65536
D_IN = 64
D_HID = 64
D_OUT = 128
D_SKIP = 64

BK1 = 4096
BK2 = 4096


def _bounds_rows(ids, total):
    cols = [
        jnp.full((1, 1), jnp.sum((ids < s).astype(jnp.int32)), jnp.int32)
        for s in range(1, B)
    ]
    lt = jnp.concatenate([jnp.zeros((1, 1), jnp.int32)] + cols, axis=1)
    le = jnp.concatenate(cols + [jnp.full((1, 1), total, jnp.int32)], axis=1)
    return lt, le


def _mlp_segmax_body(xb, posb, ball, w1a, w1b, b1r, w2r, b2r, out_ref,
                     bnd, bnd_sm, hbuf):
    h1 = jnp.dot(xb[...], w1a[...], preferred_element_type=jnp.float32)
    h1 = h1 + jnp.dot(posb[...], w1b[...], preferred_element_type=jnp.float32)
    h1 = jnp.maximum(h1 + b1r[...][0][None, :], 0.0)
    h = jnp.dot(h1, w2r[...], preferred_element_type=jnp.float32)
    h = jnp.maximum(h + b2r[...][0][None, :], 0.0)

    @pl.when(pl.program_id(0) == 0)
    def _():
        ids = ball[...]
        lts = [jnp.sum((ids < s).astype(jnp.int32)) for s in range(1, B)]
        cols = [jnp.full((1, 1), v, jnp.int32) for v in lts]
        lt = jnp.concatenate([jnp.zeros((1, 1), jnp.int32)] + cols, axis=1)
        le = jnp.concatenate(cols + [jnp.full((1, 1), N, jnp.int32)], axis=1)
        bnd[0:1, :] = lt
        bnd[1:2, :] = le
        for s in range(B):
            bnd_sm[0, s] = jnp.int32(0) if s == 0 else lts[s - 1]
            bnd_sm[1, s] = jnp.int32(N) if s == B - 1 else lts[s]
        out_ref[...] = jnp.full((B, D_OUT), -jnp.inf, jnp.float32)

    hbuf[...] = h
    base = pl.program_id(0) * BK1

    hgf = jnp.max(h.reshape(BK1 // 64, 8, 8, D_OUT), axis=1)
    hgf = hgf.reshape(BK1 // 8, D_OUT)

    llo_row = jnp.clip(bnd[0:1, :] - base, 0, BK1)
    lhi_row = jnp.clip(bnd[1:2, :] - base, 0, BK1)
    glo8 = ((llo_row + 63) >> 6) << 3
    ghi8 = (lhi_row >> 6) << 3
    rr = lax.broadcasted_iota(jnp.int32, (BK1 // 8, B), 0)
    m_g = (rr >= glo8) & (rr < ghi8)

    i64 = lax.broadcasted_iota(jnp.int32, (64, 1), 0)
    rows = []
    for s in range(B):
        gmax = jnp.max(jnp.where(m_g[:, s:s + 1], hgf, -jnp.inf), axis=0,
                       keepdims=True)
        llo = jnp.clip(bnd_sm[0, s] - base, 0, BK1)
        lhi = jnp.clip(bnd_sm[1, s] - base, 0, BK1)
        slo = jnp.minimum(llo, BK1 - 64)
        shi = jnp.clip(lhi - 64, 0, BK1 - 64)
        elo = hbuf[pl.ds(slo, 64), :]
        mlo = ((i64 + slo) >= llo) & ((i64 + slo) < lhi)
        emax = jnp.max(jnp.where(mlo, elo, -jnp.inf), axis=0, keepdims=True)
        ehi = hbuf[pl.ds(shi, 64), :]
        mhi = ((i64 + shi) >= llo) & ((i64 + shi) < lhi)
        emax = jnp.maximum(
            emax,
            jnp.max(jnp.where(mhi, ehi, -jnp.inf), axis=0, keepdims=True))
        rows.append(jnp.maximum(gmax, emax))
    out_ref[...] = jnp.maximum(out_ref[...], jnp.concatenate(rows, axis=0))


def _mlp_segmax(x, pos, batch, W1, b1, W2, b2):
    w1a = W1[:D_IN]
    w1b = W1[D_IN:]
    batc = batch.astype(jnp.int32).reshape(N // 128, 128)
    grid = (N // BK1,)
    return pl.pallas_call(
        _mlp_segmax_body,
        grid=grid,
        in_specs=[
            pl.BlockSpec((BK1, D_IN), lambda i: (i, 0)),
            pl.BlockSpec((BK1, 3), lambda i: (i, 0)),
            pl.BlockSpec((N // 128, 128), lambda i: (0, 0)),
            pl.BlockSpec((D_IN, D_HID), lambda i: (0, 0)),
            pl.BlockSpec((3, D_HID), lambda i: (0, 0)),
            pl.BlockSpec((1, D_HID), lambda i: (0, 0)),
            pl.BlockSpec((D_HID, D_OUT), lambda i: (0, 0)),
            pl.BlockSpec((1, D_OUT), lambda i: (0, 0)),
        ],
        out_specs=pl.BlockSpec((B, D_OUT), lambda i: (0, 0)),
        out_shape=jax.ShapeDtypeStruct((B, D_OUT), jnp.float32),
        scratch_shapes=[pltpu.VMEM((2, B), jnp.int32),
                        pltpu.SMEM((2, B), jnp.int32),
                        pltpu.VMEM((BK1, D_OUT), jnp.float32)],
    )(x, pos, batc, w1a, w1b, b1.reshape(1, D_HID), W2, b2.reshape(1, D_OUT))


def _assemble_body(pooled_ref, bskall, xsk_ref, out_ref, bnd):
    @pl.when(pl.program_id(0) == 0)
    def _():
        lt, le = _bounds_rows(bskall[...], NSKIP)
        bnd[0:1, :] = lt
        bnd[1:2, :] = le

    r_g = (lax.broadcasted_iota(jnp.int32, (BK2, B), 0)
           + pl.program_id(0) * BK2)
    onehot = ((r_g >= bnd[0:1, :]) & (r_g < bnd[1:2, :])).astype(jnp.float32)
    g = jnp.dot(onehot, pooled_ref[...], preferred_element_type=jnp.float32)
    g = jnp.where(g < -0.5, -jnp.inf, g)
    out_ref[:, :D_OUT] = g
    out_ref[:, D_OUT:] = xsk_ref[...]


def _assemble(pooled, batch_skip, x_skip):
    bskc = batch_skip.astype(jnp.int32).reshape(NSKIP // 128, 128)
    grid = (NSKIP // BK2,)
    return pl.pallas_call(
        _assemble_body,
        grid=grid,
        in_specs=[
            pl.BlockSpec((B, D_OUT), lambda i: (0, 0)),
            pl.BlockSpec((NSKIP // 128, 128), lambda i: (0, 0)),
            pl.BlockSpec((BK2, D_SKIP), lambda i: (i, 0)),
        ],
        out_specs=pl.BlockSpec((BK2, D_OUT + D_SKIP), lambda i: (i, 0)),
        out_shape=jax.ShapeDtypeStruct((NSKIP, D_OUT + D_SKIP), jnp.float32),
        scratch_shapes=[pltpu.VMEM((2, B), jnp.int32)],
    )(pooled, bskc, x_skip)


_NC, _NS = 2, 16
_NW = _NC * _NS
_CHUNK = NSKIP // _NW
_SUB = 128
_NSUB = _CHUNK // _SUB
_NBUF = 2
_GPS = _SUB // 128


def _sc_assemble(pooled, bsk2d, x_skip):
    mesh = plsc.VectorSubcoreMesh(core_axis_name="c", subcore_axis_name="s")

    @functools.partial(
        pl.kernel,
        out_type=jax.ShapeDtypeStruct((NSKIP, D_OUT + D_SKIP), jnp.float32),
        mesh=mesh,
        scratch_types=[
            pltpu.VMEM((_CHUNK // 128, 128), jnp.int32),
            pltpu.VMEM((_NBUF, _SUB, D_OUT), jnp.float32),
            pltpu.VMEM((_NBUF, _SUB, D_SKIP), jnp.float32),
            pltpu.VMEM_SHARED((B, D_OUT), jnp.float32),
            pltpu.SemaphoreType.DMA((_NBUF,)),
            pltpu.SemaphoreType.DMA((_NBUF,)),
            pltpu.SemaphoreType.DMA((_NBUF,)),
        ],
    )
    def k(pooled_hbm, bsk_hbm, xsk_hbm, out_hbm, idx_v, gath_v, xsb_v,
          pooled_sh, gsem, xsem, wsem):
        cid = lax.axis_index("c")
        sid = lax.axis_index("s")
        wid = sid * _NC + cid
        base = wid * _CHUNK

        @pl.when(sid == 0)
        def _():
            pltpu.sync_copy(pooled_hbm, pooled_sh)

        plsc.subcore_barrier()
        pltpu.sync_copy(
            bsk_hbm.at[pl.ds(wid * (_CHUNK // 128), _CHUNK // 128)], idx_v)

        def start(j):
            bi = j % _NBUF
            b = base + j * _SUB
            for g in range(_GPS):
                r = j * _GPS + g
                pltpu.async_copy(
                    pooled_sh.at[idx_v.at[r]],
                    gath_v.at[bi, pl.ds(g * 128, 128)], gsem.at[bi])
            pltpu.async_copy(xsk_hbm.at[pl.ds(b, _SUB)], xsb_v.at[bi],
                             xsem.at[bi])

        def drain(j):
            bi = j % _NBUF
            b = base + j * _SUB
            for g in range(_GPS):
                r = j * _GPS + g
                pltpu.make_async_copy(
                    pooled_sh.at[idx_v.at[r]],
                    gath_v.at[bi, pl.ds(g * 128, 128)], gsem.at[bi]).wait()
            pltpu.make_async_copy(xsk_hbm.at[pl.ds(b, _SUB)], xsb_v.at[bi],
                                  xsem.at[bi]).wait()
            pltpu.async_copy(gath_v.at[bi],
                             out_hbm.at[pl.ds(b, _SUB), pl.ds(0, D_OUT)],
                             wsem.at[bi])
            pltpu.async_copy(xsb_v.at[bi],
                             out_hbm.at[pl.ds(b, _SUB), pl.ds(D_OUT, D_SKIP)],
                             wsem.at[bi])

        def wait_writes(j):
            bi = j % _NBUF
            b = base + j * _SUB
            pltpu.make_async_copy(
                gath_v.at[bi],
                out_hbm.at[pl.ds(b, _SUB), pl.ds(0, D_OUT)],
                wsem.at[bi]).wait()
            pltpu.make_async_copy(
                xsb_v.at[bi],
                out_hbm.at[pl.ds(b, _SUB), pl.ds(D_OUT, D_SKIP)],
                wsem.at[bi]).wait()

        start(0)
        for j in range(_NSUB):
            if j + 1 < _NSUB:
                if j + 1 >= _NBUF:
                    wait_writes(j + 1 - _NBUF)
                start(j + 1)
            drain(j)
        for j in range(_NSUB - _NBUF + 1, _NSUB):
            wait_writes(j)
        wait_writes(_NSUB - _NBUF)

    return k(pooled, bsk2d, x_skip)


def kernel(x, pos, batch, x_skip, pos_skip, batch_skip, W1, b1, W2, b2):
    pooled = _mlp_segmax(x, pos, batch, W1, b1, W2, b2)
    bsk2d = batch_skip.astype(jnp.int32).reshape(NSKIP // 128, 128)
    out = _sc_assemble(pooled, bsk2d, x_skip)
    return (out, pos_skip, batch_skip)


# speedup vs baseline: 2.7690x; 1.0033x over previous
import functools

import jax
import jax.numpy as jnp
from jax import lax
from jax.experimental import pallas as pl
from jax.experimental.pallas import tpu as pltpu
from jax.experimental.pallas import tpu_sc as plsc

B = 16
N = 16384
NSKIP = 65536
D_IN = 64
D_HID = 64
D_OUT = 128
D_SKIP = 64

BK1 = 4096
BK2 = 4096


def _bounds_rows(ids, total):
    cols = [
        jnp.full((1, 1), jnp.sum((ids < s).astype(jnp.int32)), jnp.int32)
        for s in range(1, B)
    ]
    lt = jnp.concatenate([jnp.zeros((1, 1), jnp.int32)] + cols, axis=1)
    le = jnp.concatenate(cols + [jnp.full((1, 1), total, jnp.int32)], axis=1)
    return lt, le


def _mlp_segmax_body(xb, posb, ball, w1a, w1b, b1r, w2r, b2r, out_ref,
                     bnd, bnd_sm, hbuf):
    h1 = jnp.dot(xb[...], w1a[...], preferred_element_type=jnp.float32)
    h1 = h1 + jnp.dot(posb[...], w1b[...], preferred_element_type=jnp.float32)
    h1 = jnp.maximum(h1 + b1r[...][0][None, :], 0.0)
    h = jnp.dot(h1, w2r[...], preferred_element_type=jnp.float32)
    h = jnp.maximum(h + b2r[...][0][None, :], 0.0)

    @pl.when(pl.program_id(0) == 0)
    def _():
        ids = ball[...]
        lts = [jnp.sum((ids < s).astype(jnp.int32)) for s in range(1, B)]
        cols = [jnp.full((1, 1), v, jnp.int32) for v in lts]
        lt = jnp.concatenate([jnp.zeros((1, 1), jnp.int32)] + cols, axis=1)
        le = jnp.concatenate(cols + [jnp.full((1, 1), N, jnp.int32)], axis=1)
        bnd[0:1, :] = lt
        bnd[1:2, :] = le
        for s in range(B):
            bnd_sm[0, s] = jnp.int32(0) if s == 0 else lts[s - 1]
            bnd_sm[1, s] = jnp.int32(N) if s == B - 1 else lts[s]
        out_ref[...] = jnp.full((B, D_OUT), -jnp.inf, jnp.float32)

    hbuf[...] = h
    base = pl.program_id(0) * BK1

    hgf = jnp.max(h.reshape(BK1 // 64, 8, 8, D_OUT), axis=1)
    hgf = hgf.reshape(BK1 // 8, D_OUT)

    llo_row = jnp.clip(bnd[0:1, :] - base, 0, BK1)
    lhi_row = jnp.clip(bnd[1:2, :] - base, 0, BK1)
    glo8 = ((llo_row + 63) >> 6) << 3
    ghi8 = (lhi_row >> 6) << 3
    rr = lax.broadcasted_iota(jnp.int32, (BK1 // 8, B), 0)
    m_g = (rr >= glo8) & (rr < ghi8)

    i64 = lax.broadcasted_iota(jnp.int32, (64, 1), 0)
    rows = []
    for s in range(B):
        gmax = jnp.max(jnp.where(m_g[:, s:s + 1], hgf, -jnp.inf), axis=0,
                       keepdims=True)
        llo = jnp.clip(bnd_sm[0, s] - base, 0, BK1)
        lhi = jnp.clip(bnd_sm[1, s] - base, 0, BK1)
        slo = jnp.minimum(llo, BK1 - 64)
        shi = jnp.clip(lhi - 64, 0, BK1 - 64)
        elo = hbuf[pl.ds(slo, 64), :]
        mlo = ((i64 + slo) >= llo) & ((i64 + slo) < lhi)
        emax = jnp.max(jnp.where(mlo, elo, -jnp.inf), axis=0, keepdims=True)
        ehi = hbuf[pl.ds(shi, 64), :]
        mhi = ((i64 + shi) >= llo) & ((i64 + shi) < lhi)
        emax = jnp.maximum(
            emax,
            jnp.max(jnp.where(mhi, ehi, -jnp.inf), axis=0, keepdims=True))
        rows.append(jnp.maximum(gmax, emax))
    out_ref[...] = jnp.maximum(out_ref[...], jnp.concatenate(rows, axis=0))


def _mlp_segmax(x, pos, batch, W1, b1, W2, b2):
    w1a = W1[:D_IN]
    w1b = W1[D_IN:]
    batc = batch.astype(jnp.int32).reshape(N // 128, 128)
    grid = (N // BK1,)
    return pl.pallas_call(
        _mlp_segmax_body,
        grid=grid,
        in_specs=[
            pl.BlockSpec((BK1, D_IN), lambda i: (i, 0)),
            pl.BlockSpec((BK1, 3), lambda i: (i, 0)),
            pl.BlockSpec((N // 128, 128), lambda i: (0, 0)),
            pl.BlockSpec((D_IN, D_HID), lambda i: (0, 0)),
            pl.BlockSpec((3, D_HID), lambda i: (0, 0)),
            pl.BlockSpec((1, D_HID), lambda i: (0, 0)),
            pl.BlockSpec((D_HID, D_OUT), lambda i: (0, 0)),
            pl.BlockSpec((1, D_OUT), lambda i: (0, 0)),
        ],
        out_specs=pl.BlockSpec((B, D_OUT), lambda i: (0, 0)),
        out_shape=jax.ShapeDtypeStruct((B, D_OUT), jnp.float32),
        scratch_shapes=[pltpu.VMEM((2, B), jnp.int32),
                        pltpu.SMEM((2, B), jnp.int32),
                        pltpu.VMEM((BK1, D_OUT), jnp.float32)],
    )(x, pos, batc, w1a, w1b, b1.reshape(1, D_HID), W2, b2.reshape(1, D_OUT))


def _assemble_body(pooled_ref, bskall, xsk_ref, out_ref, bnd):
    @pl.when(pl.program_id(0) == 0)
    def _():
        lt, le = _bounds_rows(bskall[...], NSKIP)
        bnd[0:1, :] = lt
        bnd[1:2, :] = le

    r_g = (lax.broadcasted_iota(jnp.int32, (BK2, B), 0)
           + pl.program_id(0) * BK2)
    onehot = ((r_g >= bnd[0:1, :]) & (r_g < bnd[1:2, :])).astype(jnp.float32)
    g = jnp.dot(onehot, pooled_ref[...], preferred_element_type=jnp.float32)
    g = jnp.where(g < -0.5, -jnp.inf, g)
    out_ref[:, :D_OUT] = g
    out_ref[:, D_OUT:] = xsk_ref[...]


def _assemble(pooled, batch_skip, x_skip):
    bskc = batch_skip.astype(jnp.int32).reshape(NSKIP // 128, 128)
    grid = (NSKIP // BK2,)
    return pl.pallas_call(
        _assemble_body,
        grid=grid,
        in_specs=[
            pl.BlockSpec((B, D_OUT), lambda i: (0, 0)),
            pl.BlockSpec((NSKIP // 128, 128), lambda i: (0, 0)),
            pl.BlockSpec((BK2, D_SKIP), lambda i: (i, 0)),
        ],
        out_specs=pl.BlockSpec((BK2, D_OUT + D_SKIP), lambda i: (i, 0)),
        out_shape=jax.ShapeDtypeStruct((NSKIP, D_OUT + D_SKIP), jnp.float32),
        scratch_shapes=[pltpu.VMEM((2, B), jnp.int32)],
    )(pooled, bskc, x_skip)


_NC, _NS = 2, 16
_NW = _NC * _NS
_CHUNK = NSKIP // _NW
_SUB = 128
_NSUB = _CHUNK // _SUB
_NBUF = 3
_GPS = _SUB // 128


def _sc_assemble(pooled, bsk2d, x_skip):
    mesh = plsc.VectorSubcoreMesh(core_axis_name="c", subcore_axis_name="s")

    @functools.partial(
        pl.kernel,
        out_type=jax.ShapeDtypeStruct((NSKIP, D_OUT + D_SKIP), jnp.float32),
        mesh=mesh,
        scratch_types=[
            pltpu.VMEM((_CHUNK // 128, 128), jnp.int32),
            pltpu.VMEM((_NBUF, _SUB, D_OUT), jnp.float32),
            pltpu.VMEM((_NBUF, _SUB, D_SKIP), jnp.float32),
            pltpu.VMEM_SHARED((B, D_OUT), jnp.float32),
            pltpu.SemaphoreType.DMA((_NBUF,)),
            pltpu.SemaphoreType.DMA((_NBUF,)),
            pltpu.SemaphoreType.DMA((_NBUF,)),
        ],
    )
    def k(pooled_hbm, bsk_hbm, xsk_hbm, out_hbm, idx_v, gath_v, xsb_v,
          pooled_sh, gsem, xsem, wsem):
        cid = lax.axis_index("c")
        sid = lax.axis_index("s")
        wid = sid * _NC + cid
        base = wid * _CHUNK

        @pl.when(sid == 0)
        def _():
            pltpu.sync_copy(pooled_hbm, pooled_sh)

        plsc.subcore_barrier()
        pltpu.sync_copy(
            bsk_hbm.at[pl.ds(wid * (_CHUNK // 128), _CHUNK // 128)], idx_v)

        def start(j):
            bi = j % _NBUF
            b = base + j * _SUB
            for g in range(_GPS):
                r = j * _GPS + g
                pltpu.async_copy(
                    pooled_sh.at[idx_v.at[r]],
                    gath_v.at[bi, pl.ds(g * 128, 128)], gsem.at[bi])
            pltpu.async_copy(xsk_hbm.at[pl.ds(b, _SUB)], xsb_v.at[bi],
                             xsem.at[bi])

        def drain(j):
            bi = j % _NBUF
            b = base + j * _SUB
            for g in range(_GPS):
                r = j * _GPS + g
                pltpu.make_async_copy(
                    pooled_sh.at[idx_v.at[r]],
                    gath_v.at[bi, pl.ds(g * 128, 128)], gsem.at[bi]).wait()
            pltpu.make_async_copy(xsk_hbm.at[pl.ds(b, _SUB)], xsb_v.at[bi],
                                  xsem.at[bi]).wait()
            pltpu.async_copy(gath_v.at[bi],
                             out_hbm.at[pl.ds(b, _SUB), pl.ds(0, D_OUT)],
                             wsem.at[bi])
            pltpu.async_copy(xsb_v.at[bi],
                             out_hbm.at[pl.ds(b, _SUB), pl.ds(D_OUT, D_SKIP)],
                             wsem.at[bi])

        def wait_writes(j):
            bi = j % _NBUF
            b = base + j * _SUB
            pltpu.make_async_copy(
                gath_v.at[bi],
                out_hbm.at[pl.ds(b, _SUB), pl.ds(0, D_OUT)],
                wsem.at[bi]).wait()
            pltpu.make_async_copy(
                xsb_v.at[bi],
                out_hbm.at[pl.ds(b, _SUB), pl.ds(D_OUT, D_SKIP)],
                wsem.at[bi]).wait()

        start(0)
        for j in range(_NSUB):
            if j + 1 < _NSUB:
                if j + 1 >= _NBUF:
                    wait_writes(j + 1 - _NBUF)
                start(j + 1)
            drain(j)
        for j in range(_NSUB - _NBUF + 1, _NSUB):
            wait_writes(j)
        wait_writes(_NSUB - _NBUF)

    return k(pooled, bsk2d, x_skip)


def kernel(x, pos, batch, x_skip, pos_skip, batch_skip, W1, b1, W2, b2):
    pooled = _mlp_segmax(x, pos, batch, W1, b1, W2, b2)
    bsk2d = batch_skip.astype(jnp.int32).reshape(NSKIP // 128, 128)
    out = _sc_assemble(pooled, bsk2d, x_skip)
    return (out, pos_skip, batch_skip)
